# baseline (speedup 1.0000x reference)
import jax
import jax.numpy as jnp
import numpy as np
from jax.experimental import pallas as pl

_NUM_SPECIES = 4
_NUM_PAIRS = 10
_RCR = 5.2
_RCA = 3.5
_ETA_R = 16.0
_ETA_A = 8.0
_ZETA_LOG2 = 5
_PAIR_AB = [(0, 0), (0, 1), (0, 2), (0, 3), (1, 1),
            (1, 2), (1, 3), (2, 2), (2, 3), (3, 3)]


def _pow_zeta(x):
    for _ in range(_ZETA_LOG2):
        x = x * x
    return x


def _iota(shape, dim):
    return jax.lax.broadcasted_iota(jnp.int32, shape, dim)


def _mol_body(sp_ref, xyz_ref, out_ref):
    A = sp_ref.shape[-1]
    JK = A * A
    HI = jax.lax.Precision.HIGHEST
    sp = sp_ref[0, 0, :]
    xyz = xyz_ref[0]

    eye = _iota((A, A), 0) == _iota((A, A), 1)

    x, y, z = xyz[:, 0], xyz[:, 1], xyz[:, 2]
    dx = x[None, :] - x[:, None]
    dy = y[None, :] - y[:, None]
    dz = z[None, :] - z[:, None]
    d2 = dx * dx + dy * dy + dz * dz
    dist = jnp.sqrt(jnp.where(eye, 1.0, d2))
    offdiag = ~eye

    fc_r = 0.5 * jnp.cos(jnp.pi * dist / _RCR) + 0.5
    mask_r = offdiag & (dist <= _RCR)
    shf_r = (0.9 + (_RCR - 0.9) / 16.0
             * _iota((1, 1, 16), 2).astype(jnp.float32))
    fc_rm = jnp.where(mask_r, fc_r, 0.0)
    rad = (0.25 * jnp.exp(-_ETA_R * (dist[:, :, None] - shf_r) ** 2)
           * fc_rm[:, :, None])
    oh_s = (sp[:, None]
            == _iota((1, _NUM_SPECIES), 1)).astype(jnp.float32)
    radial = jnp.einsum('ijr,js->isr', rad, oh_s,
                        preferred_element_type=jnp.float32)

    fc_a = 0.5 * jnp.cos(jnp.pi * dist / _RCA) + 0.5
    mask_a = offdiag & (dist <= _RCA)
    fc_am = jnp.where(mask_a, fc_a, 0.0)

    E1 = (_iota((A, JK), 1) // A == _iota((A, JK), 0)).astype(jnp.float32)
    E2 = (_iota((A, JK), 1) % A == _iota((A, JK), 0)).astype(jnp.float32)
    d_1 = jnp.dot(dist, E1, precision=HI)
    d_2 = jnp.dot(dist, E2, precision=HI)
    fc1 = jnp.dot(fc_am, E1, precision=HI)
    fc2 = jnp.dot(fc_am, E2, precision=HI)

    E1T = (_iota((JK, A), 0) // A == _iota((JK, A), 1)).astype(jnp.float32)
    E2T = (_iota((JK, A), 0) % A == _iota((JK, A), 1)).astype(jnp.float32)
    d2rows = jnp.dot(E1T, d2, precision=HI)
    d2jk = jnp.sum(d2rows * E2T, axis=1)
    inner = 0.5 * (d_1 * d_1 + d_2 * d_2 - d2jk)
    denom = jnp.maximum(d_1 * d_2, 1e-10)
    c95 = 0.95 * jnp.clip(inner / denom, -1.0, 1.0)
    sin_t = jnp.sqrt(1.0 - c95 * c95)
    avg = 0.5 * (d_1 + d_2)
    lane = _iota((1, JK), 1)
    jlk = ((lane // A) < (lane % A)).astype(jnp.float32)
    gate2 = 2.0 * fc1 * fc2 * jlk

    spf = sp.astype(jnp.float32)[:, None]
    s1 = jnp.dot(E1T, spf)
    s2 = jnp.dot(E2T, spf)
    cols = []
    for (a, b) in _PAIR_AB:
        w = jnp.where((s1 == float(a)) & (s2 == float(b)), 1.0, 0.0)
        if a != b:
            w = w + jnp.where((s1 == float(b)) & (s2 == float(a)), 1.0, 0.0)
        cols.append(w)
    oh_p = jnp.concatenate(cols, axis=1)

    f1s = []
    for zi in range(8):
        shz = np.pi * (zi + 0.5) / 8.0
        czv, szv = float(np.cos(shz)), float(np.sin(shz))
        f1s.append(_pow_zeta(0.5 * (1.0 + c95 * czv + sin_t * szv)))
    outs = []
    for a in range(4):
        sha = 0.9 + (_RCA - 0.9) / 4.0 * a
        f2g = jnp.exp(-_ETA_A * (avg - sha) ** 2) * gate2
        for zi in range(8):
            outs.append(jnp.dot(f2g * f1s[zi], oh_p,
                                preferred_element_type=jnp.float32))
    angt = jnp.concatenate(outs, axis=1)
    r_i = _iota((320, 320), 0)
    c_i = _iota((320, 320), 1)
    P = ((r_i % 10) * 32 + r_i // 10 == c_i).astype(jnp.float32)
    ang = jnp.dot(angt, P, precision=HI)

    out_ref[0] = jnp.concatenate(
        [radial.reshape(A, _NUM_SPECIES * 16), ang], axis=1)


def kernel(species, coordinates):
    N, A = species.shape
    sp32 = species.astype(jnp.int32).reshape(N, 1, A)
    aev = pl.pallas_call(
        _mol_body,
        grid=(N,),
        in_specs=[
            pl.BlockSpec((1, 1, A), lambda n: (n, 0, 0)),
            pl.BlockSpec((1, A, 3), lambda n: (n, 0, 0)),
        ],
        out_specs=pl.BlockSpec((1, A, 384), lambda n: (n, 0, 0)),
        out_shape=jax.ShapeDtypeStruct((N, A, 384), jnp.float32),
    )(sp32, coordinates)
    return (species, aev)


# speedup vs baseline: 1.0079x; 1.0079x over previous
import jax
import jax.numpy as jnp
import numpy as np
from jax.experimental import pallas as pl

_NUM_SPECIES = 4
_NUM_PAIRS = 10
_RCR = 5.2
_RCA = 3.5
_ETA_R = 16.0
_ETA_A = 8.0
_ZETA_LOG2 = 5
_PAIR_AB = [(0, 0), (0, 1), (0, 2), (0, 3), (1, 1),
            (1, 2), (1, 3), (2, 2), (2, 3), (3, 3)]


def _pow_zeta(x):
    for _ in range(_ZETA_LOG2):
        x = x * x
    return x


def _iota(shape, dim):
    return jax.lax.broadcasted_iota(jnp.int32, shape, dim)


def _mol_body(sp_ref, xyz_ref, e1_ref, e2_ref, e1t_ref, e2t_ref, p_ref,
              out_ref):
    A = sp_ref.shape[-1]
    JK = A * A
    HI = jax.lax.Precision.HIGHEST
    sp = sp_ref[0, 0, :]
    xyz = xyz_ref[0]
    E1 = e1_ref[...]
    E2 = e2_ref[...]
    E1T = e1t_ref[...]
    E2T = e2t_ref[...]
    P = p_ref[...]

    eye = _iota((A, A), 0) == _iota((A, A), 1)

    x, y, z = xyz[:, 0], xyz[:, 1], xyz[:, 2]
    dx = x[None, :] - x[:, None]
    dy = y[None, :] - y[:, None]
    dz = z[None, :] - z[:, None]
    d2 = dx * dx + dy * dy + dz * dz
    dist = jnp.sqrt(jnp.where(eye, 1.0, d2))
    offdiag = ~eye

    fc_r = 0.5 * jnp.cos(jnp.pi * dist / _RCR) + 0.5
    mask_r = offdiag & (dist <= _RCR)
    shf_r = (0.9 + (_RCR - 0.9) / 16.0
             * _iota((1, 1, 16), 2).astype(jnp.float32))
    fc_rm = jnp.where(mask_r, fc_r, 0.0)
    rad = (0.25 * jnp.exp(-_ETA_R * (dist[:, :, None] - shf_r) ** 2)
           * fc_rm[:, :, None])
    oh_s = (sp[:, None]
            == _iota((1, _NUM_SPECIES), 1)).astype(jnp.float32)
    radial = jnp.einsum('ijr,js->isr', rad, oh_s,
                        preferred_element_type=jnp.float32)

    fc_a = 0.5 * jnp.cos(jnp.pi * dist / _RCA) + 0.5
    mask_a = offdiag & (dist <= _RCA)
    fc_am = jnp.where(mask_a, fc_a, 0.0)

    d_1 = jnp.dot(dist, E1, precision=HI)
    d_2 = jnp.dot(dist, E2, precision=HI)
    fc1 = jnp.dot(fc_am, E1, precision=HI)
    fc2 = jnp.dot(fc_am, E2, precision=HI)

    d2rows = jnp.dot(E1T, d2, precision=HI)
    d2jk = jnp.sum(d2rows * E2T, axis=1)
    inner = 0.5 * (d_1 * d_1 + d_2 * d_2 - d2jk)
    denom = jnp.maximum(d_1 * d_2, 1e-10)
    c95 = 0.95 * jnp.clip(inner / denom, -1.0, 1.0)
    sin_t = jnp.sqrt(1.0 - c95 * c95)
    avg = 0.5 * (d_1 + d_2)
    lane = _iota((1, JK), 1)
    jlk = ((lane // A) < (lane % A)).astype(jnp.float32)
    gate2 = 2.0 * fc1 * fc2 * jlk

    spf = sp.astype(jnp.float32)[:, None]
    s1 = jnp.dot(E1T, spf)
    s2 = jnp.dot(E2T, spf)
    cols = []
    for (a, b) in _PAIR_AB:
        w = jnp.where((s1 == float(a)) & (s2 == float(b)), 1.0, 0.0)
        if a != b:
            w = w + jnp.where((s1 == float(b)) & (s2 == float(a)), 1.0, 0.0)
        cols.append(w)
    oh_p = jnp.concatenate(cols, axis=1)

    f1s = []
    for zi in range(8):
        shz = np.pi * (zi + 0.5) / 8.0
        czv, szv = float(np.cos(shz)), float(np.sin(shz))
        f1s.append(_pow_zeta(0.5 * (1.0 + c95 * czv + sin_t * szv)))
    outs = []
    for a in range(4):
        sha = 0.9 + (_RCA - 0.9) / 4.0 * a
        f2g = jnp.exp(-_ETA_A * (avg - sha) ** 2) * gate2
        for zi in range(8):
            outs.append(jnp.dot(f2g * f1s[zi], oh_p,
                                preferred_element_type=jnp.float32))
    angt = jnp.concatenate(outs, axis=1)
    ang = jnp.dot(angt, P, precision=HI)

    out_ref[0] = jnp.concatenate(
        [radial.reshape(A, _NUM_SPECIES * 16), ang], axis=1)


def _expansion_constants(A):
    JK = A * A
    jk = np.arange(JK)
    e1 = (jk[None, :] // A == np.arange(A)[:, None]).astype(np.float32)
    e2 = (jk[None, :] % A == np.arange(A)[:, None]).astype(np.float32)
    r = np.arange(320)
    p = ((r[:, None] % 10) * 32 + r[:, None] // 10
         == np.arange(320)[None, :]).astype(np.float32)
    return e1, e2, e1.T.copy(), e2.T.copy(), p


def kernel(species, coordinates):
    N, A = species.shape
    sp32 = species.astype(jnp.int32).reshape(N, 1, A)
    e1, e2, e1t, e2t, p = (jnp.asarray(c) for c in _expansion_constants(A))
    JK = A * A
    const = lambda shape: pl.BlockSpec(shape, lambda n: tuple(0 for _ in shape))
    aev = pl.pallas_call(
        _mol_body,
        grid=(N,),
        in_specs=[
            pl.BlockSpec((1, 1, A), lambda n: (n, 0, 0)),
            pl.BlockSpec((1, A, 3), lambda n: (n, 0, 0)),
            const((A, JK)),
            const((A, JK)),
            const((JK, A)),
            const((JK, A)),
            const((320, 320)),
        ],
        out_specs=pl.BlockSpec((1, A, 384), lambda n: (n, 0, 0)),
        out_shape=jax.ShapeDtypeStruct((N, A, 384), jnp.float32),
    )(sp32, coordinates, e1, e2, e1t, e2t, p)
    return (species, aev)


# speedup vs baseline: 1.0847x; 1.0761x over previous
import jax
import jax.numpy as jnp
import numpy as np
from jax.experimental import pallas as pl

_NUM_SPECIES = 4
_NUM_PAIRS = 10
_RCR = 5.2
_RCA = 3.5
_ETA_R = 16.0
_ETA_A = 8.0
_ZETA_LOG2 = 5
_PAIR_AB = [(0, 0), (0, 1), (0, 2), (0, 3), (1, 1),
            (1, 2), (1, 3), (2, 2), (2, 3), (3, 3)]


def _pow_zeta(x):
    for _ in range(_ZETA_LOG2):
        x = x * x
    return x


def _iota(shape, dim):
    return jax.lax.broadcasted_iota(jnp.int32, shape, dim)


def _mol_body(sp_ref, xyz_ref, e1_ref, e2_ref, e1t_ref, e2t_ref,
              rad_out_ref, ang_out_ref):
    A = sp_ref.shape[-1]
    JK = A * A
    HI = jax.lax.Precision.HIGHEST
    sp = sp_ref[0, 0, :]
    xyz = xyz_ref[0]
    E1 = e1_ref[...]
    E2 = e2_ref[...]
    E1T = e1t_ref[...]
    E2T = e2t_ref[...]

    eye = _iota((A, A), 0) == _iota((A, A), 1)

    x, y, z = xyz[:, 0], xyz[:, 1], xyz[:, 2]
    dx = x[None, :] - x[:, None]
    dy = y[None, :] - y[:, None]
    dz = z[None, :] - z[:, None]
    d2 = dx * dx + dy * dy + dz * dz
    dist = jnp.sqrt(jnp.where(eye, 1.0, d2))
    offdiag = ~eye

    fc_r = 0.5 * jnp.cos(jnp.pi * dist / _RCR) + 0.5
    mask_r = offdiag & (dist <= _RCR)
    shf_r = (0.9 + (_RCR - 0.9) / 16.0
             * _iota((1, 1, 16), 2).astype(jnp.float32))
    fc_rm = jnp.where(mask_r, fc_r, 0.0)
    rad = (0.25 * jnp.exp(-_ETA_R * (dist[:, :, None] - shf_r) ** 2)
           * fc_rm[:, :, None])
    oh_s = (sp[:, None]
            == _iota((1, _NUM_SPECIES), 1)).astype(jnp.float32)
    radial = jnp.einsum('ijr,js->isr', rad, oh_s,
                        preferred_element_type=jnp.float32)

    fc_a = 0.5 * jnp.cos(jnp.pi * dist / _RCA) + 0.5
    mask_a = offdiag & (dist <= _RCA)
    fc_am = jnp.where(mask_a, fc_a, 0.0)

    d_1 = jnp.dot(dist, E1, precision=HI)
    d_2 = jnp.dot(dist, E2, precision=HI)
    fc1 = jnp.dot(fc_am, E1, precision=HI)
    fc2 = jnp.dot(fc_am, E2, precision=HI)

    d2rows = jnp.dot(E1T, d2, precision=HI)
    d2jk = jnp.sum(d2rows * E2T, axis=1)
    inner = 0.5 * (d_1 * d_1 + d_2 * d_2 - d2jk)
    denom = jnp.maximum(d_1 * d_2, 1e-10)
    c95 = 0.95 * jnp.clip(inner / denom, -1.0, 1.0)
    sin_t = jnp.sqrt(1.0 - c95 * c95)
    avg = 0.5 * (d_1 + d_2)
    lane = _iota((1, JK), 1)
    jlk = ((lane // A) < (lane % A)).astype(jnp.float32)
    gate2 = 2.0 * fc1 * fc2 * jlk

    spf = sp.astype(jnp.float32)[:, None]
    s1 = jnp.dot(E1T, spf)
    s2 = jnp.dot(E2T, spf)
    cols = []
    for (a, b) in _PAIR_AB:
        w = jnp.where((s1 == float(a)) & (s2 == float(b)), 1.0, 0.0)
        if a != b:
            w = w + jnp.where((s1 == float(b)) & (s2 == float(a)), 1.0, 0.0)
        cols.append(w)
    oh_p = jnp.concatenate(cols, axis=1)

    f1s = []
    for zi in range(8):
        shz = np.pi * (zi + 0.5) / 8.0
        czv, szv = float(np.cos(shz)), float(np.sin(shz))
        f1s.append(_pow_zeta(0.5 * (1.0 + c95 * czv + sin_t * szv)))
    planes = []
    for a in range(4):
        sha = 0.9 + (_RCA - 0.9) / 4.0 * a
        f2g = jnp.exp(-_ETA_A * (avg - sha) ** 2) * gate2
        for zi in range(8):
            planes.append(f2g * f1s[zi])
    T = jnp.concatenate(planes, axis=0)
    M = jnp.dot(T, oh_p, preferred_element_type=jnp.float32)

    rad_out_ref[0] = radial.reshape(A, _NUM_SPECIES * 16)
    ang_out_ref[0] = M


def _expansion_constants(A):
    JK = A * A
    jk = np.arange(JK)
    e1 = (jk[None, :] // A == np.arange(A)[:, None]).astype(np.float32)
    e2 = (jk[None, :] % A == np.arange(A)[:, None]).astype(np.float32)
    return e1, e2, e1.T.copy(), e2.T.copy()


def kernel(species, coordinates):
    N, A = species.shape
    sp32 = species.astype(jnp.int32).reshape(N, 1, A)
    e1, e2, e1t, e2t = (jnp.asarray(c) for c in _expansion_constants(A))
    JK = A * A
    const = lambda shape: pl.BlockSpec(shape, lambda n: tuple(0 for _ in shape))
    radial, ang_m = pl.pallas_call(
        _mol_body,
        grid=(N,),
        in_specs=[
            pl.BlockSpec((1, 1, A), lambda n: (n, 0, 0)),
            pl.BlockSpec((1, A, 3), lambda n: (n, 0, 0)),
            const((A, JK)),
            const((A, JK)),
            const((JK, A)),
            const((JK, A)),
        ],
        out_specs=[
            pl.BlockSpec((1, A, 64), lambda n: (n, 0, 0)),
            pl.BlockSpec((1, 32 * A, _NUM_PAIRS), lambda n: (n, 0, 0)),
        ],
        out_shape=[
            jax.ShapeDtypeStruct((N, A, 64), jnp.float32),
            jax.ShapeDtypeStruct((N, 32 * A, _NUM_PAIRS), jnp.float32),
        ],
    )(sp32, coordinates, e1, e2, e1t, e2t)
    ang = (ang_m.reshape(N, 32, A, _NUM_PAIRS)
           .transpose(0, 2, 3, 1)
           .reshape(N, A, _NUM_PAIRS * 32))
    aev = jnp.concatenate([radial, ang], axis=-1)
    return (species, aev)


# speedup vs baseline: 1.4043x; 1.2947x over previous
import jax
import jax.numpy as jnp
import numpy as np
from jax.experimental import pallas as pl

_NUM_SPECIES = 4
_NUM_PAIRS = 10
_RCR = 5.2
_RCA = 3.5
_ETA_R = 16.0
_ETA_A = 8.0
_ZETA_LOG2 = 5
_PAIR_AB = [(0, 0), (0, 1), (0, 2), (0, 3), (1, 1),
            (1, 2), (1, 3), (2, 2), (2, 3), (3, 3)]


def _pow_zeta(x):
    for _ in range(_ZETA_LOG2):
        x = x * x
    return x


def _iota(shape, dim):
    return jax.lax.broadcasted_iota(jnp.int32, shape, dim)


def _mol_body(sp_ref, xyz_ref, e1_ref, e2_ref, e1t_ref, e2t_ref,
              er_ref, ert_ref, sel4_ref, rmask_ref, sab_ref,
              rad_out_ref, ang_out_ref):
    A = sp_ref.shape[-1]
    JK = A * A
    HI = jax.lax.Precision.HIGHEST
    sp = sp_ref[0, 0, :]
    xyz = xyz_ref[0]
    E1 = e1_ref[...]
    E2 = e2_ref[...]
    E1T = e1t_ref[...]
    E2T = e2t_ref[...]
    ER = er_ref[...]
    ERT = ert_ref[...]
    SEL4 = sel4_ref[...]
    RMASK = rmask_ref[...]
    SAB = sab_ref[...]

    eye = _iota((A, A), 0) == _iota((A, A), 1)

    x, y, z = xyz[:, 0], xyz[:, 1], xyz[:, 2]
    dx = x[None, :] - x[:, None]
    dy = y[None, :] - y[:, None]
    dz = z[None, :] - z[:, None]
    d2 = dx * dx + dy * dy + dz * dz
    dist = jnp.sqrt(jnp.where(eye, 1.0, d2))
    offdiag = ~eye

    fc_r = 0.5 * jnp.cos(jnp.pi * dist / _RCR) + 0.5
    mask_r = offdiag & (dist <= _RCR)
    fc_rm = jnp.where(mask_r, fc_r, 0.0)
    dist_e = jnp.dot(dist, ER, precision=HI)
    fcm_e = jnp.dot(fc_rm, ER, precision=HI)
    shfv = (0.9 + (_RCR - 0.9) / 16.0
            * (_iota((1, 16 * A), 1) % 16).astype(jnp.float32))
    rad2 = 0.25 * jnp.exp(-_ETA_R * (dist_e - shfv) ** 2) * fcm_e
    oh_s = (sp[:, None]
            == _iota((1, _NUM_SPECIES), 1)).astype(jnp.float32)
    oh1r = jnp.dot(ERT, oh_s)
    OHR = jnp.dot(oh1r, SEL4) * RMASK
    radial = jnp.dot(rad2, OHR,
                     preferred_element_type=jnp.float32)

    fc_a = 0.5 * jnp.cos(jnp.pi * dist / _RCA) + 0.5
    mask_a = offdiag & (dist <= _RCA)
    fc_am = jnp.where(mask_a, fc_a, 0.0)

    d_1 = jnp.dot(dist, E1, precision=HI)
    d_2 = jnp.dot(dist, E2, precision=HI)
    fc1 = jnp.dot(fc_am, E1, precision=HI)
    fc2 = jnp.dot(fc_am, E2, precision=HI)

    d2rows = jnp.dot(E1T, d2, precision=HI)
    d2jk = jnp.sum(d2rows * E2T, axis=1)
    inner = 0.5 * (d_1 * d_1 + d_2 * d_2 - d2jk)
    denom = jnp.maximum(d_1 * d_2, 1e-10)
    c95 = 0.95 * jnp.clip(inner / denom, -1.0, 1.0)
    sin_t = jnp.sqrt(1.0 - c95 * c95)
    avg = 0.5 * (d_1 + d_2)
    lane = _iota((1, JK), 1)
    jlk = ((lane // A) < (lane % A)).astype(jnp.float32)
    gate2 = 2.0 * fc1 * fc2 * jlk

    oh1 = jnp.dot(E1T, oh_s)
    oh2 = jnp.dot(E2T, oh_s)
    SA, SB, SB2 = SAB[0:4], SAB[4:8], SAB[8:12]
    oh_p = (jnp.dot(oh1, SA) * jnp.dot(oh2, SB)
            + jnp.dot(oh1, SB2) * jnp.dot(oh2, SA))

    f1s = []
    for zi in range(8):
        shz = np.pi * (zi + 0.5) / 8.0
        czv, szv = float(np.cos(shz)), float(np.sin(shz))
        f1s.append(_pow_zeta(0.5 * (1.0 + c95 * czv + sin_t * szv)))
    planes = []
    for a in range(4):
        sha = 0.9 + (_RCA - 0.9) / 4.0 * a
        f2g = jnp.exp(-_ETA_A * (avg - sha) ** 2) * gate2
        for zi in range(8):
            planes.append(f2g * f1s[zi])
    T = jnp.concatenate(planes, axis=0)
    M = jnp.dot(T, oh_p, preferred_element_type=jnp.float32)

    rad_out_ref[0] = radial
    ang_out_ref[0] = M


def _expansion_constants(A):
    JK = A * A
    jk = np.arange(JK)
    e1 = (jk[None, :] // A == np.arange(A)[:, None]).astype(np.float32)
    e2 = (jk[None, :] % A == np.arange(A)[:, None]).astype(np.float32)
    jr = np.arange(16 * A)
    er = (jr[None, :] // 16 == np.arange(A)[:, None]).astype(np.float32)
    c64 = np.arange(64)
    sel4 = (c64[None, :] // 16 == np.arange(4)[:, None]).astype(np.float32)
    rmask = (jr[:, None] % 16 == c64[None, :] % 16).astype(np.float32)
    sa = np.zeros((4, 10), np.float32)
    sb = np.zeros((4, 10), np.float32)
    sb2 = np.zeros((4, 10), np.float32)
    for p, (a, b) in enumerate(_PAIR_AB):
        sa[a, p] = 1.0
        sb[b, p] = 1.0
        if a != b:
            sb2[b, p] = 1.0
    sab = np.concatenate([sa, sb, sb2], axis=0)
    return e1, e2, e1.T.copy(), e2.T.copy(), er, er.T.copy(), sel4, rmask, sab


def kernel(species, coordinates):
    N, A = species.shape
    sp32 = species.astype(jnp.int32).reshape(N, 1, A)
    consts = [jnp.asarray(c) for c in _expansion_constants(A)]
    const = lambda shape: pl.BlockSpec(shape, lambda n: tuple(0 for _ in shape))
    radial, ang_m = pl.pallas_call(
        _mol_body,
        grid=(N,),
        in_specs=[
            pl.BlockSpec((1, 1, A), lambda n: (n, 0, 0)),
            pl.BlockSpec((1, A, 3), lambda n: (n, 0, 0)),
        ] + [const(c.shape) for c in consts],
        out_specs=[
            pl.BlockSpec((1, A, 64), lambda n: (n, 0, 0)),
            pl.BlockSpec((1, 32 * A, _NUM_PAIRS), lambda n: (n, 0, 0)),
        ],
        out_shape=[
            jax.ShapeDtypeStruct((N, A, 64), jnp.float32),
            jax.ShapeDtypeStruct((N, 32 * A, _NUM_PAIRS), jnp.float32),
        ],
    )(sp32, coordinates, *consts)
    ang = (ang_m.reshape(N, 32, A, _NUM_PAIRS)
           .transpose(0, 2, 3, 1)
           .reshape(N, A, _NUM_PAIRS * 32))
    aev = jnp.concatenate([radial, ang], axis=-1)
    return (species, aev)


# speedup vs baseline: 1.6293x; 1.1602x over previous
import jax
import jax.numpy as jnp
import numpy as np
from jax.experimental import pallas as pl

_NUM_SPECIES = 4
_NUM_PAIRS = 10
_RCR = 5.2
_RCA = 3.5
_ETA_R = 16.0
_ETA_A = 8.0
_ZETA_LOG2 = 5
_PAIR_AB = [(0, 0), (0, 1), (0, 2), (0, 3), (1, 1),
            (1, 2), (1, 3), (2, 2), (2, 3), (3, 3)]


def _pow_zeta(x):
    for _ in range(_ZETA_LOG2):
        x = x * x
    return x


def _iota(shape, dim):
    return jax.lax.broadcasted_iota(jnp.int32, shape, dim)


_B = 4


def _mol_body(sp_ref, xyz_ref, e1_ref, e2_ref, e1t_ref, e2t_ref,
              er_ref, ert_ref, sel4_ref, rmask_ref, sab_ref,
              rad_out_ref, ang_out_ref):
    A = sp_ref.shape[-1]
    JK = A * A
    HI = jax.lax.Precision.HIGHEST
    E1 = e1_ref[...]
    E2 = e2_ref[...]
    E1T = e1t_ref[...]
    E2T = e2t_ref[...]
    ER = er_ref[...]
    ERT = ert_ref[...]
    SEL4 = sel4_ref[...]
    RMASK = rmask_ref[...]
    SAB = sab_ref[...]

    for b in range(_B):
        radial, M = _one_mol(sp_ref[b, 0, :], xyz_ref[b], E1, E2, E1T, E2T,
                             ER, ERT, SEL4, RMASK, SAB)
        rad_out_ref[b] = radial
        ang_out_ref[b] = M


def _one_mol(sp, xyz, E1, E2, E1T, E2T, ER, ERT, SEL4, RMASK, SAB):
    A = sp.shape[-1]
    JK = A * A
    HI = jax.lax.Precision.HIGHEST

    eye = _iota((A, A), 0) == _iota((A, A), 1)

    x, y, z = xyz[:, 0], xyz[:, 1], xyz[:, 2]
    dx = x[None, :] - x[:, None]
    dy = y[None, :] - y[:, None]
    dz = z[None, :] - z[:, None]
    d2 = dx * dx + dy * dy + dz * dz
    dist = jnp.sqrt(jnp.where(eye, 1.0, d2))
    offdiag = ~eye

    fc_r = 0.5 * jnp.cos(jnp.pi * dist / _RCR) + 0.5
    mask_r = offdiag & (dist <= _RCR)
    fc_rm = jnp.where(mask_r, fc_r, 0.0)
    dist_e = jnp.dot(dist, ER, precision=HI)
    fcm_e = jnp.dot(fc_rm, ER, precision=HI)
    shfv = (0.9 + (_RCR - 0.9) / 16.0
            * (_iota((1, 16 * A), 1) % 16).astype(jnp.float32))
    rad2 = 0.25 * jnp.exp(-_ETA_R * (dist_e - shfv) ** 2) * fcm_e
    oh_s = (sp[:, None]
            == _iota((1, _NUM_SPECIES), 1)).astype(jnp.float32)
    oh1r = jnp.dot(ERT, oh_s)
    OHR = jnp.dot(oh1r, SEL4) * RMASK
    radial = jnp.dot(rad2, OHR,
                     preferred_element_type=jnp.float32)

    fc_a = 0.5 * jnp.cos(jnp.pi * dist / _RCA) + 0.5
    mask_a = offdiag & (dist <= _RCA)
    fc_am = jnp.where(mask_a, fc_a, 0.0)

    d_1 = jnp.dot(dist, E1, precision=HI)
    d_2 = jnp.dot(dist, E2, precision=HI)
    fc1 = jnp.dot(fc_am, E1, precision=HI)
    fc2 = jnp.dot(fc_am, E2, precision=HI)

    d2rows = jnp.dot(E1T, d2, precision=HI)
    d2jk = jnp.sum(d2rows * E2T, axis=1)
    inner = 0.5 * (d_1 * d_1 + d_2 * d_2 - d2jk)
    denom = jnp.maximum(d_1 * d_2, 1e-10)
    c95 = 0.95 * jnp.clip(inner / denom, -1.0, 1.0)
    sin_t = jnp.sqrt(1.0 - c95 * c95)
    avg = 0.5 * (d_1 + d_2)
    lane = _iota((1, JK), 1)
    jlk = ((lane // A) < (lane % A)).astype(jnp.float32)
    gate2 = 2.0 * fc1 * fc2 * jlk

    oh1 = jnp.dot(E1T, oh_s)
    oh2 = jnp.dot(E2T, oh_s)
    SA, SB, SB2 = SAB[0:4], SAB[4:8], SAB[8:12]
    oh_p = (jnp.dot(oh1, SA) * jnp.dot(oh2, SB)
            + jnp.dot(oh1, SB2) * jnp.dot(oh2, SA))

    f1s = []
    for zi in range(8):
        shz = np.pi * (zi + 0.5) / 8.0
        czv, szv = float(np.cos(shz)), float(np.sin(shz))
        f1s.append(_pow_zeta(0.5 * (1.0 + c95 * czv + sin_t * szv)))
    planes = []
    for a in range(4):
        sha = 0.9 + (_RCA - 0.9) / 4.0 * a
        f2g = jnp.exp(-_ETA_A * (avg - sha) ** 2) * gate2
        for zi in range(8):
            planes.append(f2g * f1s[zi])
    T = jnp.concatenate(planes, axis=0)
    M = jnp.dot(T, oh_p, preferred_element_type=jnp.float32)

    return radial, M


def _expansion_constants(A):
    JK = A * A
    jk = np.arange(JK)
    e1 = (jk[None, :] // A == np.arange(A)[:, None]).astype(np.float32)
    e2 = (jk[None, :] % A == np.arange(A)[:, None]).astype(np.float32)
    jr = np.arange(16 * A)
    er = (jr[None, :] // 16 == np.arange(A)[:, None]).astype(np.float32)
    c64 = np.arange(64)
    sel4 = (c64[None, :] // 16 == np.arange(4)[:, None]).astype(np.float32)
    rmask = (jr[:, None] % 16 == c64[None, :] % 16).astype(np.float32)
    sa = np.zeros((4, 10), np.float32)
    sb = np.zeros((4, 10), np.float32)
    sb2 = np.zeros((4, 10), np.float32)
    for p, (a, b) in enumerate(_PAIR_AB):
        sa[a, p] = 1.0
        sb[b, p] = 1.0
        if a != b:
            sb2[b, p] = 1.0
    sab = np.concatenate([sa, sb, sb2], axis=0)
    return e1, e2, e1.T.copy(), e2.T.copy(), er, er.T.copy(), sel4, rmask, sab


def kernel(species, coordinates):
    N, A = species.shape
    sp32 = species.astype(jnp.int32).reshape(N, 1, A)
    consts = [jnp.asarray(c) for c in _expansion_constants(A)]
    const = lambda shape: pl.BlockSpec(shape, lambda n: tuple(0 for _ in shape))
    radial, ang_m = pl.pallas_call(
        _mol_body,
        grid=(N // _B,),
        in_specs=[
            pl.BlockSpec((_B, 1, A), lambda n: (n, 0, 0)),
            pl.BlockSpec((_B, A, 3), lambda n: (n, 0, 0)),
        ] + [const(c.shape) for c in consts],
        out_specs=[
            pl.BlockSpec((_B, A, 64), lambda n: (n, 0, 0)),
            pl.BlockSpec((_B, 32 * A, _NUM_PAIRS), lambda n: (n, 0, 0)),
        ],
        out_shape=[
            jax.ShapeDtypeStruct((N, A, 64), jnp.float32),
            jax.ShapeDtypeStruct((N, 32 * A, _NUM_PAIRS), jnp.float32),
        ],
    )(sp32, coordinates, *consts)
    ang = (ang_m.reshape(N, 32, A, _NUM_PAIRS)
           .transpose(0, 2, 3, 1)
           .reshape(N, A, _NUM_PAIRS * 32))
    aev = jnp.concatenate([radial, ang], axis=-1)
    return (species, aev)


# speedup vs baseline: 1.6931x; 1.0392x over previous
import jax
import jax.numpy as jnp
import numpy as np
from jax.experimental import pallas as pl

_NUM_SPECIES = 4
_NUM_PAIRS = 10
_RCR = 5.2
_RCA = 3.5
_ETA_R = 16.0
_ETA_A = 8.0
_ZETA_LOG2 = 5
_PAIR_AB = [(0, 0), (0, 1), (0, 2), (0, 3), (1, 1),
            (1, 2), (1, 3), (2, 2), (2, 3), (3, 3)]


def _pow_zeta(x):
    for _ in range(_ZETA_LOG2):
        x = x * x
    return x


def _iota(shape, dim):
    return jax.lax.broadcasted_iota(jnp.int32, shape, dim)


_B = 8


def _mol_body(sp_ref, xyz_ref, e1_ref, e2_ref, e1t_ref, e2t_ref,
              er_ref, ert_ref, sel4_ref, rmask_ref, sab_ref,
              rad_out_ref, ang_out_ref):
    A = sp_ref.shape[-1]
    JK = A * A
    HI = jax.lax.Precision.HIGHEST
    E1 = e1_ref[...]
    E2 = e2_ref[...]
    E1T = e1t_ref[...]
    E2T = e2t_ref[...]
    ER = er_ref[...]
    ERT = ert_ref[...]
    SEL4 = sel4_ref[...]
    RMASK = rmask_ref[...]
    SAB = sab_ref[...]

    for b in range(_B):
        radial, M = _one_mol(sp_ref[b, 0, :], xyz_ref[b], E1, E2, E1T, E2T,
                             ER, ERT, SEL4, RMASK, SAB)
        rad_out_ref[b] = radial
        ang_out_ref[b] = M


def _one_mol(sp, xyz, E1, E2, E1T, E2T, ER, ERT, SEL4, RMASK, SAB):
    A = sp.shape[-1]
    JK = A * A
    HI = jax.lax.Precision.HIGHEST

    eye = _iota((A, A), 0) == _iota((A, A), 1)

    x, y, z = xyz[:, 0], xyz[:, 1], xyz[:, 2]
    dx = x[None, :] - x[:, None]
    dy = y[None, :] - y[:, None]
    dz = z[None, :] - z[:, None]
    d2 = dx * dx + dy * dy + dz * dz
    dist = jnp.sqrt(jnp.where(eye, 1.0, d2))
    offdiag = ~eye

    fc_r = 0.5 * jnp.cos(jnp.pi * dist / _RCR) + 0.5
    mask_r = offdiag & (dist <= _RCR)
    fc_rm = jnp.where(mask_r, fc_r, 0.0)
    dist_e = jnp.dot(dist, ER, precision=HI)
    fcm_e = jnp.dot(fc_rm, ER, precision=HI)
    shfv = (0.9 + (_RCR - 0.9) / 16.0
            * (_iota((1, 16 * A), 1) % 16).astype(jnp.float32))
    rad2 = 0.25 * jnp.exp(-_ETA_R * (dist_e - shfv) ** 2) * fcm_e
    oh_s = (sp[:, None]
            == _iota((1, _NUM_SPECIES), 1)).astype(jnp.float32)
    oh1r = jnp.dot(ERT, oh_s)
    OHR = jnp.dot(oh1r, SEL4) * RMASK
    radial = jnp.dot(rad2, OHR,
                     preferred_element_type=jnp.float32)

    fc_a = 0.5 * jnp.cos(jnp.pi * dist / _RCA) + 0.5
    mask_a = offdiag & (dist <= _RCA)
    fc_am = jnp.where(mask_a, fc_a, 0.0)

    d_1 = jnp.dot(dist, E1, precision=HI)
    d_2 = jnp.dot(dist, E2, precision=HI)
    fc1 = jnp.dot(fc_am, E1, precision=HI)
    fc2 = jnp.dot(fc_am, E2, precision=HI)

    d2rows = jnp.dot(E1T, d2, precision=HI)
    d2jk = jnp.sum(d2rows * E2T, axis=1)
    inner = 0.5 * (d_1 * d_1 + d_2 * d_2 - d2jk)
    denom = jnp.maximum(d_1 * d_2, 1e-10)
    c95 = 0.95 * jnp.clip(inner / denom, -1.0, 1.0)
    sin_t = jnp.sqrt(1.0 - c95 * c95)
    avg = 0.5 * (d_1 + d_2)
    lane = _iota((1, JK), 1)
    jlk = ((lane // A) < (lane % A)).astype(jnp.float32)
    gate2 = 2.0 * fc1 * fc2 * jlk

    oh1 = jnp.dot(E1T, oh_s)
    oh2 = jnp.dot(E2T, oh_s)
    SA, SB, SB2 = SAB[0:4], SAB[4:8], SAB[8:12]
    oh_p = (jnp.dot(oh1, SA) * jnp.dot(oh2, SB)
            + jnp.dot(oh1, SB2) * jnp.dot(oh2, SA))

    f1s = []
    for zi in range(8):
        shz = np.pi * (zi + 0.5) / 8.0
        czv, szv = float(np.cos(shz)), float(np.sin(shz))
        f1s.append(_pow_zeta(0.5 * (1.0 + c95 * czv + sin_t * szv)))
    planes = []
    for a in range(4):
        sha = 0.9 + (_RCA - 0.9) / 4.0 * a
        f2g = jnp.exp(-_ETA_A * (avg - sha) ** 2) * gate2
        for zi in range(8):
            planes.append(f2g * f1s[zi])
    T = jnp.concatenate(planes, axis=0)
    M = jnp.dot(T, oh_p, preferred_element_type=jnp.float32)

    return radial, M


def _expansion_constants(A):
    JK = A * A
    jk = np.arange(JK)
    e1 = (jk[None, :] // A == np.arange(A)[:, None]).astype(np.float32)
    e2 = (jk[None, :] % A == np.arange(A)[:, None]).astype(np.float32)
    jr = np.arange(16 * A)
    er = (jr[None, :] // 16 == np.arange(A)[:, None]).astype(np.float32)
    c64 = np.arange(64)
    sel4 = (c64[None, :] // 16 == np.arange(4)[:, None]).astype(np.float32)
    rmask = (jr[:, None] % 16 == c64[None, :] % 16).astype(np.float32)
    sa = np.zeros((4, 10), np.float32)
    sb = np.zeros((4, 10), np.float32)
    sb2 = np.zeros((4, 10), np.float32)
    for p, (a, b) in enumerate(_PAIR_AB):
        sa[a, p] = 1.0
        sb[b, p] = 1.0
        if a != b:
            sb2[b, p] = 1.0
    sab = np.concatenate([sa, sb, sb2], axis=0)
    return e1, e2, e1.T.copy(), e2.T.copy(), er, er.T.copy(), sel4, rmask, sab


def kernel(species, coordinates):
    N, A = species.shape
    sp32 = species.astype(jnp.int32).reshape(N, 1, A)
    consts = [jnp.asarray(c) for c in _expansion_constants(A)]
    const = lambda shape: pl.BlockSpec(shape, lambda n: tuple(0 for _ in shape))
    radial, ang_m = pl.pallas_call(
        _mol_body,
        grid=(N // _B,),
        in_specs=[
            pl.BlockSpec((_B, 1, A), lambda n: (n, 0, 0)),
            pl.BlockSpec((_B, A, 3), lambda n: (n, 0, 0)),
        ] + [const(c.shape) for c in consts],
        out_specs=[
            pl.BlockSpec((_B, A, 64), lambda n: (n, 0, 0)),
            pl.BlockSpec((_B, 32 * A, _NUM_PAIRS), lambda n: (n, 0, 0)),
        ],
        out_shape=[
            jax.ShapeDtypeStruct((N, A, 64), jnp.float32),
            jax.ShapeDtypeStruct((N, 32 * A, _NUM_PAIRS), jnp.float32),
        ],
    )(sp32, coordinates, *consts)
    ang = (ang_m.reshape(N, 32, A, _NUM_PAIRS)
           .transpose(0, 2, 3, 1)
           .reshape(N, A, _NUM_PAIRS * 32))
    aev = jnp.concatenate([radial, ang], axis=-1)
    return (species, aev)


# speedup vs baseline: 1.7255x; 1.0192x over previous
import jax
import jax.numpy as jnp
import numpy as np
from jax.experimental import pallas as pl

_NUM_SPECIES = 4
_NUM_PAIRS = 10
_RCR = 5.2
_RCA = 3.5
_ETA_R = 16.0
_ETA_A = 8.0
_ZETA_LOG2 = 5
_PAIR_AB = [(0, 0), (0, 1), (0, 2), (0, 3), (1, 1),
            (1, 2), (1, 3), (2, 2), (2, 3), (3, 3)]


def _pow_zeta(x):
    for _ in range(_ZETA_LOG2):
        x = x * x
    return x


def _iota(shape, dim):
    return jax.lax.broadcasted_iota(jnp.int32, shape, dim)


_B = 16


def _mol_body(sp_ref, xyz_ref, e1_ref, e2_ref, e1t_ref, e2t_ref,
              er_ref, ert_ref, sel4_ref, rmask_ref, sab_ref,
              rad_out_ref, ang_out_ref):
    A = sp_ref.shape[-1]
    JK = A * A
    HI = jax.lax.Precision.HIGHEST
    E1 = e1_ref[...]
    E2 = e2_ref[...]
    E1T = e1t_ref[...]
    E2T = e2t_ref[...]
    ER = er_ref[...]
    ERT = ert_ref[...]
    SEL4 = sel4_ref[...]
    RMASK = rmask_ref[...]
    SAB = sab_ref[...]

    for b in range(_B):
        radial, M = _one_mol(sp_ref[b, 0, :], xyz_ref[b], E1, E2, E1T, E2T,
                             ER, ERT, SEL4, RMASK, SAB)
        rad_out_ref[b] = radial
        ang_out_ref[b] = M


def _one_mol(sp, xyz, E1, E2, E1T, E2T, ER, ERT, SEL4, RMASK, SAB):
    A = sp.shape[-1]
    JK = A * A
    HI = jax.lax.Precision.HIGHEST

    eye = _iota((A, A), 0) == _iota((A, A), 1)

    x, y, z = xyz[:, 0], xyz[:, 1], xyz[:, 2]
    dx = x[None, :] - x[:, None]
    dy = y[None, :] - y[:, None]
    dz = z[None, :] - z[:, None]
    d2 = dx * dx + dy * dy + dz * dz
    dist = jnp.sqrt(jnp.where(eye, 1.0, d2))
    offdiag = ~eye

    fc_r = 0.5 * jnp.cos(jnp.pi * dist / _RCR) + 0.5
    mask_r = offdiag & (dist <= _RCR)
    fc_rm = jnp.where(mask_r, fc_r, 0.0)
    dist_e = jnp.dot(dist, ER, precision=HI)
    fcm_e = jnp.dot(fc_rm, ER, precision=HI)
    shfv = (0.9 + (_RCR - 0.9) / 16.0
            * (_iota((1, 16 * A), 1) % 16).astype(jnp.float32))
    rad2 = 0.25 * jnp.exp(-_ETA_R * (dist_e - shfv) ** 2) * fcm_e
    oh_s = (sp[:, None]
            == _iota((1, _NUM_SPECIES), 1)).astype(jnp.float32)
    oh1r = jnp.dot(ERT, oh_s)
    OHR = jnp.dot(oh1r, SEL4) * RMASK
    radial = jnp.dot(rad2, OHR,
                     preferred_element_type=jnp.float32)

    fc_a = 0.5 * jnp.cos(jnp.pi * dist / _RCA) + 0.5
    mask_a = offdiag & (dist <= _RCA)
    fc_am = jnp.where(mask_a, fc_a, 0.0)

    d_1 = jnp.dot(dist, E1, precision=HI)
    d_2 = jnp.dot(dist, E2, precision=HI)
    fc1 = jnp.dot(fc_am, E1, precision=HI)
    fc2 = jnp.dot(fc_am, E2, precision=HI)

    d2rows = jnp.dot(E1T, d2, precision=HI)
    d2jk = jnp.sum(d2rows * E2T, axis=1)
    inner = 0.5 * (d_1 * d_1 + d_2 * d_2 - d2jk)
    denom = jnp.maximum(d_1 * d_2, 1e-10)
    c95 = 0.95 * jnp.clip(inner / denom, -1.0, 1.0)
    sin_t = jnp.sqrt(1.0 - c95 * c95)
    avg = 0.5 * (d_1 + d_2)
    lane = _iota((1, JK), 1)
    jlk = ((lane // A) < (lane % A)).astype(jnp.float32)
    gate2 = 2.0 * fc1 * fc2 * jlk

    oh1 = jnp.dot(E1T, oh_s)
    oh2 = jnp.dot(E2T, oh_s)
    SA, SB, SB2 = SAB[0:4], SAB[4:8], SAB[8:12]
    oh_p = (jnp.dot(oh1, SA) * jnp.dot(oh2, SB)
            + jnp.dot(oh1, SB2) * jnp.dot(oh2, SA))

    f1s = []
    for zi in range(8):
        shz = np.pi * (zi + 0.5) / 8.0
        czv, szv = float(np.cos(shz)), float(np.sin(shz))
        f1s.append(_pow_zeta(0.5 * (1.0 + c95 * czv + sin_t * szv)))
    planes = []
    for a in range(4):
        sha = 0.9 + (_RCA - 0.9) / 4.0 * a
        f2g = jnp.exp(-_ETA_A * (avg - sha) ** 2) * gate2
        for zi in range(8):
            planes.append(f2g * f1s[zi])
    T = jnp.concatenate(planes, axis=0)
    M = jnp.dot(T, oh_p, preferred_element_type=jnp.float32)

    return radial, M


def _expansion_constants(A):
    JK = A * A
    jk = np.arange(JK)
    e1 = (jk[None, :] // A == np.arange(A)[:, None]).astype(np.float32)
    e2 = (jk[None, :] % A == np.arange(A)[:, None]).astype(np.float32)
    jr = np.arange(16 * A)
    er = (jr[None, :] // 16 == np.arange(A)[:, None]).astype(np.float32)
    c64 = np.arange(64)
    sel4 = (c64[None, :] // 16 == np.arange(4)[:, None]).astype(np.float32)
    rmask = (jr[:, None] % 16 == c64[None, :] % 16).astype(np.float32)
    sa = np.zeros((4, 10), np.float32)
    sb = np.zeros((4, 10), np.float32)
    sb2 = np.zeros((4, 10), np.float32)
    for p, (a, b) in enumerate(_PAIR_AB):
        sa[a, p] = 1.0
        sb[b, p] = 1.0
        if a != b:
            sb2[b, p] = 1.0
    sab = np.concatenate([sa, sb, sb2], axis=0)
    return e1, e2, e1.T.copy(), e2.T.copy(), er, er.T.copy(), sel4, rmask, sab


def kernel(species, coordinates):
    N, A = species.shape
    sp32 = species.astype(jnp.int32).reshape(N, 1, A)
    consts = [jnp.asarray(c) for c in _expansion_constants(A)]
    const = lambda shape: pl.BlockSpec(shape, lambda n: tuple(0 for _ in shape))
    radial, ang_m = pl.pallas_call(
        _mol_body,
        grid=(N // _B,),
        in_specs=[
            pl.BlockSpec((_B, 1, A), lambda n: (n, 0, 0)),
            pl.BlockSpec((_B, A, 3), lambda n: (n, 0, 0)),
        ] + [const(c.shape) for c in consts],
        out_specs=[
            pl.BlockSpec((_B, A, 64), lambda n: (n, 0, 0)),
            pl.BlockSpec((_B, 32 * A, _NUM_PAIRS), lambda n: (n, 0, 0)),
        ],
        out_shape=[
            jax.ShapeDtypeStruct((N, A, 64), jnp.float32),
            jax.ShapeDtypeStruct((N, 32 * A, _NUM_PAIRS), jnp.float32),
        ],
    )(sp32, coordinates, *consts)
    ang = (ang_m.reshape(N, 32, A, _NUM_PAIRS)
           .transpose(0, 2, 3, 1)
           .reshape(N, A, _NUM_PAIRS * 32))
    aev = jnp.concatenate([radial, ang], axis=-1)
    return (species, aev)
